# baseline (speedup 1.0000x reference)
import jax
import jax.numpy as jnp
import numpy as np
from jax.experimental import pallas as pl
from jax.experimental.pallas import tpu as pltpu

BINS_N = 10
MU_C = 0.02
MU2_C = MU_C * MU_C
MMT_C = 0.7
ALPHA_C = 0.25
EPS_C = 1e-5

B_N, H_N, W_N, C_N = 64, 256, 256, 5
N_PLANES = 3 + 2 * (BINS_N - 1)


def _edge_list():
    e = [float(x) / BINS_N for x in range(BINS_N + 1)]
    e[-1] = 1000.0
    return [np.float32(v) for v in e]


STRIP = 8
N_STRIPS = H_N // STRIP


def _plane_kernel(p_ref, t_ref, out_ref):
    i = pl.program_id(0)
    edges = _edge_list()

    def fold(x):
        return jnp.sum(x.reshape(STRIP, W_N // 128, 128), axis=1)

    def strip_body(j, accs):
        sl = pl.ds(j * STRIP, STRIP)
        pc = p_ref[0, sl, :]
        tc = t_ref[0, sl, :]

        u = 2.0 * tc - 1.0
        one_m_t = 1.0 - tc
        x_t = pc * u + one_m_t
        alpha_t = ALPHA_C * u + one_m_t
        om = 1.0 - x_t
        fl = -alpha_t * om * om * jnp.log(x_t + EPS_C)

        v = jnp.where(tc > 0.1, 1.0, 0.0)

        loss = []
        g = []
        for c in range(1, C_N):
            dpc = p_ref[c, sl, :] - t_ref[c, sl, :]
            root = jnp.sqrt(dpc * dpc + MU2_C)
            loss.append(root - MU_C)
            g.append(jnp.abs(dpc / root))

        wsum0 = (loss[0] + loss[1]) + (loss[2] + loss[3])
        new = [accs[0] + fold(fl),
               accs[1] + fold(v),
               accs[2] + fold(v * wsum0)]
        q = 3
        for b in range(1, BINS_N):
            m = [gc >= edges[b] for gc in g]
            mf = [jnp.where(mc, 1.0, 0.0) for mc in m]
            cnt = (mf[0] + mf[1]) + (mf[2] + mf[3])
            wv = [jnp.where(mc, lc, 0.0) for mc, lc in zip(m, loss)]
            ws = (wv[0] + wv[1]) + (wv[2] + wv[3])
            new.append(accs[q] + fold(v * cnt))
            new.append(accs[q + 1] + fold(v * ws))
            q += 2
        return tuple(new)

    zero = jnp.zeros((STRIP, 128), jnp.float32)
    accs = jax.lax.fori_loop(
        0, N_STRIPS, strip_body, tuple(zero for _ in range(N_PLANES)))
    vals = jnp.stack(accs)

    @pl.when(i == 0)
    def _():
        out_ref[...] = vals

    @pl.when(i > 0)
    def _():
        out_ref[...] += vals


def _streaming_pass(p3d, t3d):
    return pl.pallas_call(
        _plane_kernel,
        grid=(B_N,),
        in_specs=[
            pl.BlockSpec((C_N, H_N, W_N), lambda i: (i, 0, 0)),
            pl.BlockSpec((C_N, H_N, W_N), lambda i: (i, 0, 0)),
        ],
        out_specs=pl.BlockSpec((N_PLANES, 8, 128), lambda i: (0, 0, 0)),
        out_shape=jax.ShapeDtypeStruct((N_PLANES, 8, 128), jnp.float32),
        compiler_params=pltpu.CompilerParams(
            dimension_semantics=("arbitrary",),
        ),
    )(p3d, t3d)


@jax.jit
def kernel(preds, targets):
    p3d = jnp.transpose(preds, (0, 3, 1, 2)).reshape(B_N * C_N, H_N, W_N)
    t3d = jnp.transpose(targets, (0, 3, 1, 2)).reshape(B_N * C_N, H_N, W_N)
    sums = _streaming_pass(p3d, t3d).sum(axis=(1, 2))

    focal_sum = sums[0]
    tot_raw = sums[1]
    L0 = sums[2]
    S_rest = sums[3::2]
    L_rest = sums[4::2]

    tot = jnp.maximum(tot_raw, 1.0)
    S = jnp.concatenate([jnp.reshape(4.0 * tot_raw, (1,)), S_rest])
    L = jnp.concatenate([jnp.reshape(L0, (1,)), L_rest])
    counts = S - jnp.concatenate([S[1:], jnp.zeros((1,), jnp.float32)])
    lsum = L - jnp.concatenate([L[1:], jnp.zeros((1,), jnp.float32)])

    acc_sum = (1.0 - MMT_C) * counts
    n = (counts > 0).astype(jnp.float32).sum()
    per_bin_w = jnp.where(counts > 0, tot / jnp.maximum(acc_sum, 1e-12), 0.0)
    reg = (lsum * per_bin_w).sum()
    reg = jnp.where(n > 0, reg / jnp.maximum(n, 1.0), reg)
    reg_loss = reg / tot

    cls_loss = focal_sum / (B_N * H_N * W_N)
    total = cls_loss + reg_loss
    return (total,
            jax.lax.stop_gradient(reg_loss),
            jax.lax.stop_gradient(cls_loss))


# speedup vs baseline: 3.9171x; 3.9171x over previous
import jax
import jax.numpy as jnp
import numpy as np
from jax.experimental import pallas as pl
from jax.experimental.pallas import tpu as pltpu

BINS_N = 10
MU_C = 0.02
MU2_C = MU_C * MU_C
MMT_C = 0.7
ALPHA_C = 0.25
EPS_C = 1e-5

B_N, H_N, W_N, C_N = 64, 256, 256, 5
N_PLANES = 3 + 2 * (BINS_N - 1)


def _edge_list():
    e = [float(x) / BINS_N for x in range(BINS_N + 1)]
    e[-1] = 1000.0
    return [np.float32(v) for v in e]


STRIP = 8
N_STRIPS = H_N // STRIP


def _plane_kernel(p_ref, t_ref, out_ref):
    i = pl.program_id(0)
    edges = _edge_list()

    def fold(x):
        return x[:, :128] + x[:, 128:]

    def strip_body(j, accs):
        sl = pl.ds(j * STRIP, STRIP)
        pc = p_ref[0, sl, :]
        tc = t_ref[0, sl, :]

        u = 2.0 * tc - 1.0
        one_m_t = 1.0 - tc
        x_t = pc * u + one_m_t
        alpha_t = ALPHA_C * u + one_m_t
        om = 1.0 - x_t
        fl = -alpha_t * om * om * jnp.log(x_t + EPS_C)

        v = jnp.where(tc > 0.1, 1.0, 0.0)

        loss = []
        g = []
        for c in range(1, C_N):
            dpc = p_ref[c, sl, :] - t_ref[c, sl, :]
            root = jnp.sqrt(dpc * dpc + MU2_C)
            loss.append(root - MU_C)
            g.append(jnp.abs(dpc / root))

        wsum0 = (loss[0] + loss[1]) + (loss[2] + loss[3])
        new = [accs[0] + fold(fl),
               accs[1] + fold(v),
               accs[2] + fold(v * wsum0)]
        q = 3
        for b in range(1, BINS_N):
            m = [gc >= edges[b] for gc in g]
            mf = [jnp.where(mc, 1.0, 0.0) for mc in m]
            cnt = (mf[0] + mf[1]) + (mf[2] + mf[3])
            wv = [jnp.where(mc, lc, 0.0) for mc, lc in zip(m, loss)]
            ws = (wv[0] + wv[1]) + (wv[2] + wv[3])
            new.append(accs[q] + fold(v * cnt))
            new.append(accs[q + 1] + fold(v * ws))
            q += 2
        return tuple(new)

    zero = jnp.zeros((STRIP, 128), jnp.float32)
    accs = jax.lax.fori_loop(
        0, N_STRIPS, strip_body, tuple(zero for _ in range(N_PLANES)))
    vals = jnp.stack(accs)

    @pl.when(i == 0)
    def _():
        out_ref[...] = vals

    @pl.when(i > 0)
    def _():
        out_ref[...] += vals


def _streaming_pass(p3d, t3d):
    return pl.pallas_call(
        _plane_kernel,
        grid=(B_N,),
        in_specs=[
            pl.BlockSpec((C_N, H_N, W_N), lambda i: (i, 0, 0)),
            pl.BlockSpec((C_N, H_N, W_N), lambda i: (i, 0, 0)),
        ],
        out_specs=pl.BlockSpec((N_PLANES, 8, 128), lambda i: (0, 0, 0)),
        out_shape=jax.ShapeDtypeStruct((N_PLANES, 8, 128), jnp.float32),
        compiler_params=pltpu.CompilerParams(
            dimension_semantics=("arbitrary",),
        ),
    )(p3d, t3d)


@jax.jit
def kernel(preds, targets):
    p3d = jnp.transpose(preds, (0, 3, 1, 2)).reshape(B_N * C_N, H_N, W_N)
    t3d = jnp.transpose(targets, (0, 3, 1, 2)).reshape(B_N * C_N, H_N, W_N)
    sums = _streaming_pass(p3d, t3d).sum(axis=(1, 2))

    focal_sum = sums[0]
    tot_raw = sums[1]
    L0 = sums[2]
    S_rest = sums[3::2]
    L_rest = sums[4::2]

    tot = jnp.maximum(tot_raw, 1.0)
    S = jnp.concatenate([jnp.reshape(4.0 * tot_raw, (1,)), S_rest])
    L = jnp.concatenate([jnp.reshape(L0, (1,)), L_rest])
    counts = S - jnp.concatenate([S[1:], jnp.zeros((1,), jnp.float32)])
    lsum = L - jnp.concatenate([L[1:], jnp.zeros((1,), jnp.float32)])

    acc_sum = (1.0 - MMT_C) * counts
    n = (counts > 0).astype(jnp.float32).sum()
    per_bin_w = jnp.where(counts > 0, tot / jnp.maximum(acc_sum, 1e-12), 0.0)
    reg = (lsum * per_bin_w).sum()
    reg = jnp.where(n > 0, reg / jnp.maximum(n, 1.0), reg)
    reg_loss = reg / tot

    cls_loss = focal_sum / (B_N * H_N * W_N)
    total = cls_loss + reg_loss
    return (total,
            jax.lax.stop_gradient(reg_loss),
            jax.lax.stop_gradient(cls_loss))


# speedup vs baseline: 4.5489x; 1.1613x over previous
import jax
import jax.numpy as jnp
import numpy as np
from jax.experimental import pallas as pl
from jax.experimental.pallas import tpu as pltpu

BINS_N = 10
MU_C = 0.02
MU2_C = MU_C * MU_C
MMT_C = 0.7
ALPHA_C = 0.25
EPS_C = 1e-5

B_N, H_N, W_N, C_N = 64, 256, 256, 5
N_PLANES = 3 + 2 * (BINS_N - 1)


def _edge_list():
    e = [float(x) / BINS_N for x in range(BINS_N + 1)]
    e[-1] = 1000.0
    return [np.float32(v) for v in e]


STRIP = 16
N_STRIPS = H_N // STRIP


def _plane_kernel(p_ref, t_ref, out_ref):
    i = pl.program_id(0)
    edges = _edge_list()

    def fold(x):
        return x[:, :128] + x[:, 128:]

    def strip_body(j, accs):
        sl = pl.ds(j * STRIP, STRIP)
        pc = p_ref[0, sl, :]
        tc = t_ref[0, sl, :]

        u = 2.0 * tc - 1.0
        one_m_t = 1.0 - tc
        x_t = pc * u + one_m_t
        alpha_t = ALPHA_C * u + one_m_t
        om = 1.0 - x_t
        fl = -alpha_t * om * om * jnp.log(x_t + EPS_C)

        v = jnp.where(tc > 0.1, 1.0, 0.0)

        new = list(accs)
        new[0] = new[0] + fold(fl)
        new[1] = new[1] + fold(v)

        for c in range(1, C_N):
            dpc = p_ref[c, sl, :] - t_ref[c, sl, :]
            root = jnp.sqrt(dpc * dpc + MU2_C)
            lc = root - MU_C
            gc = jnp.abs(dpc / root)
            vl = v * lc
            new[2] = new[2] + fold(vl)
            q = 3
            for b in range(1, BINS_N):
                m = gc >= edges[b]
                sv = jnp.where(m, v, 0.0)
                wv = jnp.where(m, vl, 0.0)
                new[q] = new[q] + fold(sv)
                new[q + 1] = new[q + 1] + fold(wv)
                q += 2
        return tuple(new)

    zero = jnp.zeros((STRIP, 128), jnp.float32)
    accs = jax.lax.fori_loop(
        0, N_STRIPS, strip_body, tuple(zero for _ in range(N_PLANES)))
    vals = jnp.stack(accs)

    @pl.when(i == 0)
    def _():
        out_ref[...] = vals

    @pl.when(i > 0)
    def _():
        out_ref[...] += vals


def _streaming_pass(p3d, t3d):
    return pl.pallas_call(
        _plane_kernel,
        grid=(B_N,),
        in_specs=[
            pl.BlockSpec((C_N, H_N, W_N), lambda i: (i, 0, 0)),
            pl.BlockSpec((C_N, H_N, W_N), lambda i: (i, 0, 0)),
        ],
        out_specs=pl.BlockSpec((N_PLANES, STRIP, 128), lambda i: (0, 0, 0)),
        out_shape=jax.ShapeDtypeStruct((N_PLANES, STRIP, 128), jnp.float32),
        compiler_params=pltpu.CompilerParams(
            dimension_semantics=("arbitrary",),
        ),
    )(p3d, t3d)


@jax.jit
def kernel(preds, targets):
    p3d = jnp.transpose(preds, (0, 3, 1, 2)).reshape(B_N * C_N, H_N, W_N)
    t3d = jnp.transpose(targets, (0, 3, 1, 2)).reshape(B_N * C_N, H_N, W_N)
    sums = _streaming_pass(p3d, t3d).sum(axis=(1, 2))

    focal_sum = sums[0]
    tot_raw = sums[1]
    L0 = sums[2]
    S_rest = sums[3::2]
    L_rest = sums[4::2]

    tot = jnp.maximum(tot_raw, 1.0)
    S = jnp.concatenate([jnp.reshape(4.0 * tot_raw, (1,)), S_rest])
    L = jnp.concatenate([jnp.reshape(L0, (1,)), L_rest])
    counts = S - jnp.concatenate([S[1:], jnp.zeros((1,), jnp.float32)])
    lsum = L - jnp.concatenate([L[1:], jnp.zeros((1,), jnp.float32)])

    acc_sum = (1.0 - MMT_C) * counts
    n = (counts > 0).astype(jnp.float32).sum()
    per_bin_w = jnp.where(counts > 0, tot / jnp.maximum(acc_sum, 1e-12), 0.0)
    reg = (lsum * per_bin_w).sum()
    reg = jnp.where(n > 0, reg / jnp.maximum(n, 1.0), reg)
    reg_loss = reg / tot

    cls_loss = focal_sum / (B_N * H_N * W_N)
    total = cls_loss + reg_loss
    return (total,
            jax.lax.stop_gradient(reg_loss),
            jax.lax.stop_gradient(cls_loss))


# speedup vs baseline: 4.7879x; 1.0526x over previous
import jax
import jax.numpy as jnp
import numpy as np
from jax.experimental import pallas as pl
from jax.experimental.pallas import tpu as pltpu

BINS_N = 10
MU_C = 0.02
MU2_C = MU_C * MU_C
MMT_C = 0.7
ALPHA_C = 0.25
EPS_C = 1e-5

B_N, H_N, W_N, C_N = 64, 256, 256, 5
N_PLANES = 3 + 2 * (BINS_N - 1)


def _edge_list():
    e = [float(x) / BINS_N for x in range(BINS_N + 1)]
    e[-1] = 1000.0
    return [np.float32(v) for v in e]


STRIP = 16
N_STRIPS = H_N // STRIP


def _plane_kernel(p_ref, t_ref, out_ref):
    i = pl.program_id(0)
    edges = _edge_list()

    def fold(x):
        a = x[:, :128] + x[:, 128:]
        return a[:8, :] + a[8:, :]

    def strip_body(j, accs):
        sl = pl.ds(j * STRIP, STRIP)
        pc = p_ref[0, sl, :]
        tc = t_ref[0, sl, :]

        u = 2.0 * tc - 1.0
        one_m_t = 1.0 - tc
        x_t = pc * u + one_m_t
        alpha_t = ALPHA_C * u + one_m_t
        om = 1.0 - x_t
        fl = -alpha_t * om * om * jnp.log(x_t + EPS_C)

        v = jnp.where(tc > 0.1, 1.0, 0.0)

        new = list(accs)
        new[0] = new[0] + fold(fl)
        new[1] = new[1] + fold(v)

        for c in range(1, C_N):
            dpc = p_ref[c, sl, :] - t_ref[c, sl, :]
            s = dpc * dpc + MU2_C
            rs = jax.lax.rsqrt(s)
            lc = s * rs - MU_C
            vl = v * lc
            gv = jnp.abs(dpc) * rs * v
            new[2] = new[2] + fold(vl)
            q = 3
            for b in range(1, BINS_N):
                m = gv >= edges[b]
                sv = jnp.where(m, 1.0, 0.0)
                wv = jnp.where(m, vl, 0.0)
                new[q] = new[q] + fold(sv)
                new[q + 1] = new[q + 1] + fold(wv)
                q += 2
        return tuple(new)

    zero = jnp.zeros((8, 128), jnp.float32)
    accs = jax.lax.fori_loop(
        0, N_STRIPS, strip_body, tuple(zero for _ in range(N_PLANES)))
    vals = jnp.stack(accs)

    @pl.when(i == 0)
    def _():
        out_ref[...] = vals

    @pl.when(i > 0)
    def _():
        out_ref[...] += vals


def _streaming_pass(p3d, t3d):
    return pl.pallas_call(
        _plane_kernel,
        grid=(B_N,),
        in_specs=[
            pl.BlockSpec((C_N, H_N, W_N), lambda i: (i, 0, 0)),
            pl.BlockSpec((C_N, H_N, W_N), lambda i: (i, 0, 0)),
        ],
        out_specs=pl.BlockSpec((N_PLANES, 8, 128), lambda i: (0, 0, 0)),
        out_shape=jax.ShapeDtypeStruct((N_PLANES, 8, 128), jnp.float32),
        compiler_params=pltpu.CompilerParams(
            dimension_semantics=("arbitrary",),
        ),
    )(p3d, t3d)


@jax.jit
def kernel(preds, targets):
    p3d = jnp.transpose(preds, (0, 3, 1, 2)).reshape(B_N * C_N, H_N, W_N)
    t3d = jnp.transpose(targets, (0, 3, 1, 2)).reshape(B_N * C_N, H_N, W_N)
    sums = _streaming_pass(p3d, t3d).sum(axis=(1, 2))

    focal_sum = sums[0]
    tot_raw = sums[1]
    L0 = sums[2]
    S_rest = sums[3::2]
    L_rest = sums[4::2]

    tot = jnp.maximum(tot_raw, 1.0)
    S = jnp.concatenate([jnp.reshape(4.0 * tot_raw, (1,)), S_rest])
    L = jnp.concatenate([jnp.reshape(L0, (1,)), L_rest])
    counts = S - jnp.concatenate([S[1:], jnp.zeros((1,), jnp.float32)])
    lsum = L - jnp.concatenate([L[1:], jnp.zeros((1,), jnp.float32)])

    acc_sum = (1.0 - MMT_C) * counts
    n = (counts > 0).astype(jnp.float32).sum()
    per_bin_w = jnp.where(counts > 0, tot / jnp.maximum(acc_sum, 1e-12), 0.0)
    reg = (lsum * per_bin_w).sum()
    reg = jnp.where(n > 0, reg / jnp.maximum(n, 1.0), reg)
    reg_loss = reg / tot

    cls_loss = focal_sum / (B_N * H_N * W_N)
    total = cls_loss + reg_loss
    return (total,
            jax.lax.stop_gradient(reg_loss),
            jax.lax.stop_gradient(cls_loss))


# speedup vs baseline: 5.0280x; 1.0501x over previous
import jax
import jax.numpy as jnp
import numpy as np
from jax.experimental import pallas as pl
from jax.experimental.pallas import tpu as pltpu

BINS_N = 10
MU_C = 0.02
MU2_C = MU_C * MU_C
MMT_C = 0.7
ALPHA_C = 0.25
EPS_C = 1e-5

B_N, H_N, W_N, C_N = 64, 256, 256, 5
N_PLANES = 3 + 2 * (BINS_N - 1)


def _edge_list():
    e = [float(x) / BINS_N for x in range(BINS_N + 1)]
    e[-1] = 1000.0
    return [np.float32(v) for v in e]


STRIP = 16
N_STRIPS = H_N // STRIP


def _plane_kernel(p_ref, t_ref, out_ref):
    i = pl.program_id(0)
    edges = _edge_list()

    def fold(x):
        a = x[:, :128] + x[:, 128:]
        return a[:8, :] + a[8:, :]

    def one_strip(sl, accs):
        pc = p_ref[0, sl, :]
        tc = t_ref[0, sl, :]

        u = 2.0 * tc - 1.0
        one_m_t = 1.0 - tc
        x_t = pc * u + one_m_t
        alpha_t = ALPHA_C * u + one_m_t
        om = 1.0 - x_t
        fl = -alpha_t * om * om * jnp.log(x_t + EPS_C)

        v = jnp.where(tc > 0.1, 1.0, 0.0)

        new = list(accs)
        new[0] = new[0] + fold(fl)
        new[1] = new[1] + fold(v)

        for c in range(1, C_N):
            dpc = p_ref[c, sl, :] - t_ref[c, sl, :]
            s = dpc * dpc + MU2_C
            rs = jax.lax.rsqrt(s)
            lc = s * rs - MU_C
            vl = v * lc
            gv = jnp.abs(dpc) * rs * v
            new[2] = new[2] + fold(vl)
            q = 3
            for b in range(1, BINS_N):
                m = gv >= edges[b]
                sv = jnp.where(m, 1.0, 0.0)
                wv = jnp.where(m, vl, 0.0)
                new[q] = new[q] + fold(sv)
                new[q + 1] = new[q + 1] + fold(wv)
                q += 2
        return tuple(new)

    def strip_body(j, accs):
        accs = one_strip(pl.ds(j * 2 * STRIP, STRIP), accs)
        return one_strip(pl.ds((j * 2 + 1) * STRIP, STRIP), accs)

    zero = jnp.zeros((8, 128), jnp.float32)
    accs = jax.lax.fori_loop(
        0, N_STRIPS // 2, strip_body, tuple(zero for _ in range(N_PLANES)))
    vals = jnp.stack(accs)

    @pl.when(i == 0)
    def _():
        out_ref[...] = vals

    @pl.when(i > 0)
    def _():
        out_ref[...] += vals


def _streaming_pass(p3d, t3d):
    return pl.pallas_call(
        _plane_kernel,
        grid=(B_N,),
        in_specs=[
            pl.BlockSpec((C_N, H_N, W_N), lambda i: (i, 0, 0)),
            pl.BlockSpec((C_N, H_N, W_N), lambda i: (i, 0, 0)),
        ],
        out_specs=pl.BlockSpec((N_PLANES, 8, 128), lambda i: (0, 0, 0)),
        out_shape=jax.ShapeDtypeStruct((N_PLANES, 8, 128), jnp.float32),
        compiler_params=pltpu.CompilerParams(
            dimension_semantics=("arbitrary",),
        ),
    )(p3d, t3d)


@jax.jit
def kernel(preds, targets):
    p3d = jnp.transpose(preds, (0, 3, 1, 2)).reshape(B_N * C_N, H_N, W_N)
    t3d = jnp.transpose(targets, (0, 3, 1, 2)).reshape(B_N * C_N, H_N, W_N)
    sums = _streaming_pass(p3d, t3d).sum(axis=(1, 2))

    focal_sum = sums[0]
    tot_raw = sums[1]
    L0 = sums[2]
    S_rest = sums[3::2]
    L_rest = sums[4::2]

    tot = jnp.maximum(tot_raw, 1.0)
    S = jnp.concatenate([jnp.reshape(4.0 * tot_raw, (1,)), S_rest])
    L = jnp.concatenate([jnp.reshape(L0, (1,)), L_rest])
    counts = S - jnp.concatenate([S[1:], jnp.zeros((1,), jnp.float32)])
    lsum = L - jnp.concatenate([L[1:], jnp.zeros((1,), jnp.float32)])

    acc_sum = (1.0 - MMT_C) * counts
    n = (counts > 0).astype(jnp.float32).sum()
    per_bin_w = jnp.where(counts > 0, tot / jnp.maximum(acc_sum, 1e-12), 0.0)
    reg = (lsum * per_bin_w).sum()
    reg = jnp.where(n > 0, reg / jnp.maximum(n, 1.0), reg)
    reg_loss = reg / tot

    cls_loss = focal_sum / (B_N * H_N * W_N)
    total = cls_loss + reg_loss
    return (total,
            jax.lax.stop_gradient(reg_loss),
            jax.lax.stop_gradient(cls_loss))


# speedup vs baseline: 5.0931x; 1.0129x over previous
import jax
import jax.numpy as jnp
import numpy as np
from jax.experimental import pallas as pl
from jax.experimental.pallas import tpu as pltpu

BINS_N = 10
MU_C = 0.02
MU2_C = MU_C * MU_C
MMT_C = 0.7
ALPHA_C = 0.25
EPS_C = 1e-5

B_N, H_N, W_N, C_N = 64, 256, 256, 5
N_PLANES = 3 + 2 * (BINS_N - 1)


def _edge_list():
    e = [float(x) / BINS_N for x in range(BINS_N + 1)]
    e[-1] = 1000.0
    return [np.float32(v) for v in e]


STRIP = 16
N_STRIPS = H_N // STRIP


def _plane_kernel(p_ref, t_ref, out_ref):
    i = pl.program_id(0)
    edges = _edge_list()

    def fold(x):
        a = x[:, :128] + x[:, 128:]
        return a[:8, :] + a[8:, :]

    def one_strip(sl, accs):
        pc = p_ref[0, sl, :]
        tc = t_ref[0, sl, :]

        u = 2.0 * tc - 1.0
        one_m_t = 1.0 - tc
        x_t = pc * u + one_m_t
        alpha_t = ALPHA_C * u + one_m_t
        om = 1.0 - x_t
        fl = -alpha_t * om * om * jnp.log(x_t + EPS_C)

        v = jnp.where(tc > 0.1, 1.0, 0.0)

        new = list(accs)
        new[0] = new[0] + fold(fl)
        new[1] = new[1] + fold(v)

        for c in range(1, C_N):
            dpc = p_ref[c, sl, :] - t_ref[c, sl, :]
            s = dpc * dpc + MU2_C
            rs = jax.lax.rsqrt(s)
            lc = s * rs - MU_C
            vl = v * lc
            gv = jnp.abs(dpc) * rs * v
            new[2] = new[2] + fold(vl)
            q = 3
            for b in range(1, BINS_N):
                m = gv >= edges[b]
                sv = jnp.where(m, 1.0, 0.0)
                wv = jnp.where(m, vl, 0.0)
                new[q] = new[q] + fold(sv)
                new[q + 1] = new[q + 1] + fold(wv)
                q += 2
        return tuple(new)

    def strip_body(j, accs):
        for u in range(4):
            accs = one_strip(pl.ds((j * 4 + u) * STRIP, STRIP), accs)
        return accs

    zero = jnp.zeros((8, 128), jnp.float32)
    accs = jax.lax.fori_loop(
        0, N_STRIPS // 4, strip_body, tuple(zero for _ in range(N_PLANES)))
    vals = jnp.stack(accs)

    @pl.when(i == 0)
    def _():
        out_ref[...] = vals

    @pl.when(i > 0)
    def _():
        out_ref[...] += vals


def _streaming_pass(p3d, t3d):
    return pl.pallas_call(
        _plane_kernel,
        grid=(B_N,),
        in_specs=[
            pl.BlockSpec((C_N, H_N, W_N), lambda i: (i, 0, 0)),
            pl.BlockSpec((C_N, H_N, W_N), lambda i: (i, 0, 0)),
        ],
        out_specs=pl.BlockSpec((N_PLANES, 8, 128), lambda i: (0, 0, 0)),
        out_shape=jax.ShapeDtypeStruct((N_PLANES, 8, 128), jnp.float32),
        compiler_params=pltpu.CompilerParams(
            dimension_semantics=("arbitrary",),
        ),
    )(p3d, t3d)


@jax.jit
def kernel(preds, targets):
    p3d = jnp.transpose(preds, (0, 3, 1, 2)).reshape(B_N * C_N, H_N, W_N)
    t3d = jnp.transpose(targets, (0, 3, 1, 2)).reshape(B_N * C_N, H_N, W_N)
    sums = _streaming_pass(p3d, t3d).sum(axis=(1, 2))

    focal_sum = sums[0]
    tot_raw = sums[1]
    L0 = sums[2]
    S_rest = sums[3::2]
    L_rest = sums[4::2]

    tot = jnp.maximum(tot_raw, 1.0)
    S = jnp.concatenate([jnp.reshape(4.0 * tot_raw, (1,)), S_rest])
    L = jnp.concatenate([jnp.reshape(L0, (1,)), L_rest])
    counts = S - jnp.concatenate([S[1:], jnp.zeros((1,), jnp.float32)])
    lsum = L - jnp.concatenate([L[1:], jnp.zeros((1,), jnp.float32)])

    acc_sum = (1.0 - MMT_C) * counts
    n = (counts > 0).astype(jnp.float32).sum()
    per_bin_w = jnp.where(counts > 0, tot / jnp.maximum(acc_sum, 1e-12), 0.0)
    reg = (lsum * per_bin_w).sum()
    reg = jnp.where(n > 0, reg / jnp.maximum(n, 1.0), reg)
    reg_loss = reg / tot

    cls_loss = focal_sum / (B_N * H_N * W_N)
    total = cls_loss + reg_loss
    return (total,
            jax.lax.stop_gradient(reg_loss),
            jax.lax.stop_gradient(cls_loss))
